# Initial kernel scaffold; baseline (speedup 1.0000x reference)
import jax
import jax.numpy as jnp
from jax.experimental import pallas as pl


def kernel(embed_states, batch_index, W_dev, b_dev, W_act, b_act):
    raise NotImplementedError("write your pallas kernel here")


# speedup vs baseline: 1.8042x; 1.8042x over previous
import functools

import jax
import jax.numpy as jnp
from jax.experimental import pallas as pl
from jax.experimental.pallas import tpu as pltpu

N, E, A, B = 16384, 2048, 512, 16
RB = 512
NB = N // RB
AP = A + 128
_NEG = -1e30


def _pass_a(x_ref, w_ref, b_ref, bi_ref, out_ref, bm_ref, bs_ref):
    x = x_ref[...].astype(jnp.bfloat16)
    acts = jax.lax.dot_general(
        x, w_ref[...], (((1,), (0,)), ((), ())),
        preferred_element_type=jnp.float32)
    acts = acts + b_ref[...]
    act = acts[:, :A]
    dsv = acts[:, A:A + 1]
    rowmax = jnp.max(act, axis=1, keepdims=True)
    rlse = rowmax + jnp.log(
        jnp.sum(jnp.exp(act - rowmax), axis=1, keepdims=True))
    out_ref[...] = act - rlse + dsv
    bi = bi_ref[0]
    oh = bi == jax.lax.broadcasted_iota(jnp.int32, (RB, B), 1)
    mb = jnp.max(jnp.where(oh, dsv, _NEG), axis=0)
    sb = jnp.sum(jnp.where(oh, jnp.exp(dsv - mb[None, :]), 0.0), axis=0)
    bm_ref[...] = mb.reshape(1, 1, B)
    bs_ref[...] = sb.reshape(1, 1, B)


def _pass_c(part_ref, bm_ref, bs_ref, bi_ref, out_ref):
    bm = bm_ref[0]
    bs = bs_ref[0]
    m = jnp.max(bm, axis=0)
    s = jnp.sum(bs * jnp.exp(bm - m[None, :]), axis=0)
    lse = m + jnp.log(s)
    bi = bi_ref[0]
    oh = bi == jax.lax.broadcasted_iota(jnp.int32, (RB, B), 1)
    lse_row = jnp.sum(jnp.where(oh, lse[None, :], 0.0), axis=1,
                      keepdims=True)
    out_ref[...] = part_ref[...] - lse_row


def kernel(embed_states, batch_index, W_dev, b_dev, W_act, b_act):
    wc = jnp.concatenate([W_act, W_dev], axis=1)
    wc = jnp.pad(wc, ((0, 0), (0, AP - (A + 1)))).astype(jnp.bfloat16)
    bc = jnp.pad(jnp.concatenate([b_act, b_dev]), (0, AP - (A + 1)))
    bc = bc.reshape(1, AP)
    bi3 = batch_index.reshape(NB, RB, 1)

    part, bm, bs = pl.pallas_call(
        _pass_a,
        grid=(NB,),
        in_specs=[
            pl.BlockSpec((RB, E), lambda i: (i, 0)),
            pl.BlockSpec((E, AP), lambda i: (0, 0)),
            pl.BlockSpec((1, AP), lambda i: (0, 0)),
            pl.BlockSpec((1, RB, 1), lambda i: (i, 0, 0)),
        ],
        out_specs=[
            pl.BlockSpec((RB, A), lambda i: (i, 0)),
            pl.BlockSpec((1, 1, B), lambda i: (i, 0, 0)),
            pl.BlockSpec((1, 1, B), lambda i: (i, 0, 0)),
        ],
        out_shape=[
            jax.ShapeDtypeStruct((N, A), jnp.float32),
            jax.ShapeDtypeStruct((NB, 1, B), jnp.float32),
            jax.ShapeDtypeStruct((NB, 1, B), jnp.float32),
        ],
        compiler_params=pltpu.CompilerParams(
            dimension_semantics=("parallel",)),
    )(embed_states, wc, bc, bi3)

    bm2 = bm.reshape(1, NB, B)
    bs2 = bs.reshape(1, NB, B)
    out = pl.pallas_call(
        _pass_c,
        grid=(NB,),
        in_specs=[
            pl.BlockSpec((RB, A), lambda i: (i, 0)),
            pl.BlockSpec((1, NB, B), lambda i: (0, 0, 0)),
            pl.BlockSpec((1, NB, B), lambda i: (0, 0, 0)),
            pl.BlockSpec((1, RB, 1), lambda i: (i, 0, 0)),
        ],
        out_specs=pl.BlockSpec((RB, A), lambda i: (i, 0)),
        out_shape=jax.ShapeDtypeStruct((N, A), jnp.float32),
        input_output_aliases={0: 0},
        compiler_params=pltpu.CompilerParams(
            dimension_semantics=("parallel",)),
    )(part, bm2, bs2, bi3)
    return out
